# baseline (speedup 1.0000x reference)
import jax
import jax.numpy as jnp
from jax.experimental import pallas as pl
from jax.experimental.pallas import tpu as pltpu

B, S, D, E = 4, 8192, 2048, 64
RCH = 512
NCHKS = B * S // RCH
CPB = NCHKS // B


def _tc_sum_kernel(x_ref, out_ref):
    out_ref[...] = jnp.sum(x_ref[...], axis=0)[None, None]


def _combine_kernel(p_ref, w_ref, b_ref, out_ref):
    s = jnp.sum(p_ref[...], axis=1)
    logits = jax.lax.dot_general(
        s, w_ref[...],
        dimension_numbers=(((1,), (1,)), ((), ())),
        preferred_element_type=jnp.float32,
    ) + b_ref[...]
    out_ref[...] = jnp.argmax(logits, axis=1).astype(jnp.int32)[None, :]


def kernel(x, W, b):
    partials = pl.pallas_call(
        _tc_sum_kernel,
        grid=(NCHKS,),
        in_specs=[pl.BlockSpec((RCH, D), lambda i: (i, 0))],
        out_specs=pl.BlockSpec((1, 1, D), lambda i: (i, 0, 0)),
        out_shape=jax.ShapeDtypeStruct((NCHKS, 1, D), jnp.float32),
    )(x.reshape(B * S, D))
    out = pl.pallas_call(
        _combine_kernel,
        in_specs=[
            pl.BlockSpec((B, CPB, D), lambda: (0, 0, 0)),
            pl.BlockSpec((E, D), lambda: (0, 0)),
            pl.BlockSpec((1, E), lambda: (0, 0)),
        ],
        out_specs=pl.BlockSpec((1, B), lambda: (0, 0)),
        out_shape=jax.ShapeDtypeStruct((1, B), jnp.int32),
    )(partials.reshape(B, CPB, D), W, b.reshape(1, E))
    return out.reshape(B)


# speedup vs baseline: 1.0904x; 1.0904x over previous
import jax
import jax.numpy as jnp
from jax import lax
from jax.experimental import pallas as pl
from jax.experimental.pallas import tpu as pltpu

B, S, D, E = 4, 8192, 2048, 64
RB = 256
NCH = B * S // RB
CPB = S // RB
NBUF = 4
NG = NCH // NBUF


def _tc_kernel(x_hbm, w_ref, b_ref, out_ref, b0, b1, b2, b3, acc_ref,
               s0, s1, s2, s3):
    bufs = [b0, b1, b2, b3]
    sems = [s0, s1, s2, s3]

    def start(c, k):
        pltpu.make_async_copy(
            x_hbm.at[pl.ds(c * RB, RB)], bufs[k], sems[k]).start()

    def wait(k):
        pltpu.make_async_copy(
            x_hbm.at[pl.ds(0, RB)], bufs[k], sems[k]).wait()

    acc_ref[...] = jnp.zeros_like(acc_ref)
    for k in range(NBUF):
        start(k, k)

    def loop(g, _):
        for k in range(NBUF):
            c = NBUF * g + k
            wait(k)
            part = jnp.sum(bufs[k][...].reshape(RB // 8, 8, D), axis=0)
            batch = c // CPB
            acc_ref[pl.ds(batch * 8, 8), :] += part

            @pl.when(c + NBUF < NCH)
            def _next():
                start(c + NBUF, k)

        return 0

    lax.fori_loop(0, NG, loop, 0)

    s = jnp.sum(acc_ref[...].reshape(B, 8, D), axis=1)
    logits = jax.lax.dot_general(
        s, w_ref[...],
        dimension_numbers=(((1,), (1,)), ((), ())),
        preferred_element_type=jnp.float32,
    ) + b_ref[...]
    out_ref[...] = jnp.argmax(logits, axis=1).astype(jnp.int32)[None, :]


def kernel(x, W, b):
    out = pl.pallas_call(
        _tc_kernel,
        in_specs=[
            pl.BlockSpec(memory_space=pltpu.MemorySpace.HBM),
            pl.BlockSpec(memory_space=pltpu.MemorySpace.VMEM),
            pl.BlockSpec(memory_space=pltpu.MemorySpace.VMEM),
        ],
        out_specs=pl.BlockSpec(memory_space=pltpu.MemorySpace.VMEM),
        out_shape=jax.ShapeDtypeStruct((1, B), jnp.int32),
        scratch_shapes=[
            pltpu.VMEM((RB, D), jnp.float32),
            pltpu.VMEM((RB, D), jnp.float32),
            pltpu.VMEM((RB, D), jnp.float32),
            pltpu.VMEM((RB, D), jnp.float32),
            pltpu.VMEM((B * 8, D), jnp.float32),
            pltpu.SemaphoreType.DMA,
            pltpu.SemaphoreType.DMA,
            pltpu.SemaphoreType.DMA,
            pltpu.SemaphoreType.DMA,
        ],
    )(x.reshape(B * S, D), W, b.reshape(1, E))
    return out.reshape(B)
